# Initial kernel scaffold; baseline (speedup 1.0000x reference)
import functools

import jax
import jax.numpy as jnp
from jax import lax
from jax.experimental import pallas as pl
from jax.experimental.pallas import tpu as pltpu
from jax.experimental.pallas import tpu_sc as plsc

N_NODES = 100000
Q = 16
DT = 0.1

NC = 2
NS = 16
NW = NC * NS
SUB = 128
B = 1024
KSUB = B // SUB

N1 = 100352
RPS = N1 // NS

E_PAD = 3211264
E_PER_W = E_PAD // NW
NCHUNK = E_PER_W // B


def _sc_body(f_hbm, src_hbm, dst_hbm, w_hbm, z16_hbm, z4_hbm,
             acc_out, sd_out,
             srcv, dstv, wv, rows, vals, acc_sh, sd_sh, gsem):
    c = lax.axis_index("c")
    s = lax.axis_index("s")
    wid = s * NC + c

    r0 = s * RPS
    pltpu.sync_copy(z16_hbm.at[pl.ds(r0, RPS)], acc_sh.at[pl.ds(r0, RPS)])
    pltpu.sync_copy(z4_hbm.at[pl.ds(r0, RPS)], sd_sh.at[pl.ds(r0, RPS)])
    plsc.subcore_barrier()

    ebase = wid * E_PER_W
    rbase = ebase // SUB

    iota = lax.iota(jnp.int32, 16)
    lane_mod4 = iota & 3
    m_w = lane_mod4 == 0
    pat = jnp.where(lane_mod4 == 1, 1.0, 0.0).astype(jnp.float32)
    quad = iota >> 2
    zeros16_i = jnp.zeros((16,), jnp.int32)

    @pl.loop(0, NCHUNK)
    def _chunk(ci):
        rb = rbase + ci * KSUB
        eb = ebase + ci * B
        pltpu.sync_copy(src_hbm.at[pl.ds(rb, KSUB)], srcv)
        pltpu.sync_copy(dst_hbm.at[pl.ds(rb, KSUB)], dstv)
        pltpu.sync_copy(w_hbm.at[pl.ds(eb, B)], wv)

        handles = []
        for j in range(KSUB):
            handles.append(pltpu.async_copy(
                f_hbm.at[srcv.at[j]], rows.at[pl.ds(j * SUB, SUB)], gsem))
        for h in handles:
            h.wait()

        @pl.loop(0, B)
        def _mul(j):
            wb = plsc.load_gather(wv, [zeros16_i + j])
            rows[j] = rows[j] * wb

        @pl.loop(0, B // 4)
        def _mkval(g):
            wg = plsc.load_gather(wv, [quad + g * 4])
            val = jnp.where(m_w, wg, pat)
            plsc.store_scatter(vals, [quad + g * 4, lane_mod4], val)

        for j in range(KSUB):
            pltpu.sync_copy(rows.at[pl.ds(j * SUB, SUB)],
                            acc_sh.at[dstv.at[j]], add=True)
            pltpu.sync_copy(vals.at[pl.ds(j * SUB, SUB)],
                            sd_sh.at[dstv.at[j]], add=True)

    plsc.subcore_barrier()
    pltpu.sync_copy(acc_sh.at[pl.ds(r0, RPS)], acc_out.at[c, pl.ds(r0, RPS)])
    pltpu.sync_copy(sd_sh.at[pl.ds(r0, RPS)], sd_out.at[c, pl.ds(r0, RPS)])


@jax.jit
def _sc_segment_sums(f, src2, dst2, w1):
    z16 = jnp.zeros((N1, Q), jnp.float32)
    z4 = jnp.zeros((N1, 4), jnp.float32)
    mesh = plsc.VectorSubcoreMesh(core_axis_name="c", subcore_axis_name="s")
    k = pl.kernel(
        _sc_body,
        out_type=[jax.ShapeDtypeStruct((NC, N1, Q), jnp.float32),
                  jax.ShapeDtypeStruct((NC, N1, 4), jnp.float32)],
        mesh=mesh,
        scratch_types=[
            pltpu.VMEM((KSUB, SUB), jnp.int32),
            pltpu.VMEM((KSUB, SUB), jnp.int32),
            pltpu.VMEM((B,), jnp.float32),
            pltpu.VMEM((B, Q), jnp.float32),
            pltpu.VMEM((B, 4), jnp.float32),
            pltpu.VMEM_SHARED((N1, Q), jnp.float32),
            pltpu.VMEM_SHARED((N1, 4), jnp.float32),
            pltpu.SemaphoreType.DMA,
        ],
    )
    return k(f, src2, dst2, w1, z16, z4)


def _combine_body(f_ref, coll_ref, srcterm_ref, acc_ref, sd_ref, xi_ref,
                  out_ref):
    f = f_ref[...]
    a = acc_ref[0] + acc_ref[1]
    sv = sd_ref[0, :, 0:1] + sd_ref[1, :, 0:1]
    deg = sd_ref[0, :, 1:2] + sd_ref[1, :, 1:2]
    deg = jnp.maximum(deg, 1.0)
    xi = xi_ref[...]
    transport = xi * (a - f * sv) / deg
    out_ref[...] = f - DT * (transport - coll_ref[...] + srcterm_ref[...])


@jax.jit
def _tc_combine(f, coll, srcterm, acc, sd, xi):
    R = 1000
    grid = (N_NODES // R,)
    return pl.pallas_call(
        _combine_body,
        grid=grid,
        in_specs=[
            pl.BlockSpec((R, Q), lambda i: (i, 0)),
            pl.BlockSpec((R, Q), lambda i: (i, 0)),
            pl.BlockSpec((R, Q), lambda i: (i, 0)),
            pl.BlockSpec((NC, R, Q), lambda i: (0, i, 0)),
            pl.BlockSpec((NC, R, 4), lambda i: (0, i, 0)),
            pl.BlockSpec((1, Q), lambda i: (0, 0)),
        ],
        out_specs=pl.BlockSpec((R, Q), lambda i: (i, 0)),
        out_shape=jax.ShapeDtypeStruct((N_NODES, Q), jnp.float32),
    )(f, coll, srcterm, acc, sd, xi)


def kernel(f_distribution, collision_term, source_term, edge_index,
           edge_weight, xi_velocities):
    E = edge_weight.shape[0]
    pad = E_PAD - E
    src = jnp.concatenate([edge_index[0], jnp.zeros((pad,), jnp.int32)])
    dst = jnp.concatenate([edge_index[1],
                           jnp.full((pad,), N_NODES, jnp.int32)])
    w = jnp.concatenate([edge_weight, jnp.zeros((pad,), jnp.float32)])
    src2 = src.reshape(E_PAD // SUB, SUB)
    dst2 = dst.reshape(E_PAD // SUB, SUB)
    acc, sd = _sc_segment_sums(f_distribution, src2, dst2, w)
    return _tc_combine(f_distribution, collision_term, source_term, acc, sd,
                       xi_velocities.reshape(1, Q))


# speedup vs baseline: 39.2200x; 39.2200x over previous
import dataclasses
import functools

import jax
import jax.numpy as jnp
from jax import lax
from jax.experimental import pallas as pl
from jax.experimental.pallas import tpu as pltpu
from jax.experimental.pallas import tpu_sc as plsc

N_NODES = 100000
Q = 16
DT = 0.1

NC = 2
NS = 16
NW = NC * NS
SUB = 128
B = 1024
KSUB = B // SUB

N1 = 100352
RPS = N1 // NS

E_PAD = 3211264
E_PER_W = E_PAD // NW
NCHUNK = E_PER_W // B


def _acc_body(f_hbm, src_hbm, dst_hbm, w_hbm,
              acc_out,
              srcv, dstv, wv, rows, acc_sh, gsem):
    c = lax.axis_index("c")
    s = lax.axis_index("s")
    wid = s * NC + c

    zeros16_i = jnp.zeros((16,), jnp.int32)
    zeros16_f = jnp.zeros((16,), jnp.float32)

    @pl.loop(0, B)
    def _zrow(j):
        rows[j] = zeros16_f

    r0 = pl.multiple_of(s * RPS, 8)
    for t in range((RPS + B - 1) // B):
        nr = min(B, RPS - t * B)
        pltpu.sync_copy(rows.at[pl.ds(0, nr)],
                        acc_sh.at[pl.ds(r0 + t * B, nr)])
    plsc.subcore_barrier()

    ebase = wid * E_PER_W
    rbase = ebase // SUB

    @pl.loop(0, NCHUNK)
    def _chunk(ci):
        rb = pl.multiple_of(rbase + ci * KSUB, 8)
        eb = pl.multiple_of(ebase + ci * B, 8)
        pltpu.sync_copy(src_hbm.at[pl.ds(rb, KSUB)], srcv)
        pltpu.sync_copy(dst_hbm.at[pl.ds(rb, KSUB)], dstv)
        pltpu.sync_copy(w_hbm.at[pl.ds(eb, B)], wv)

        handles = []
        for j in range(KSUB):
            handles.append(pltpu.async_copy(
                f_hbm.at[srcv.at[j]], rows.at[pl.ds(j * SUB, SUB)], gsem))
        for h in handles:
            h.wait()

        @pl.loop(0, B)
        def _mul(j):
            wb = plsc.load_gather(wv, [zeros16_i + j])
            rows[j] = rows[j] * wb

        for j in range(KSUB):
            pltpu.sync_copy(rows.at[pl.ds(j * SUB, SUB)],
                            acc_sh.at[dstv.at[j]], add=True)

    plsc.subcore_barrier()
    pltpu.sync_copy(acc_sh.at[pl.ds(r0, RPS)], acc_out.at[c, pl.ds(r0, RPS)])


def _sd_body(dst_hbm, w_hbm,
             sd_out,
             dstv, wv, vals, sd_sh, gsem):
    c = lax.axis_index("c")
    s = lax.axis_index("s")
    wid = s * NC + c

    iota = lax.iota(jnp.int32, 16)
    pat = jnp.where(iota == 1, 1.0, 0.0).astype(jnp.float32)
    zeros16_i = jnp.zeros((16,), jnp.int32)
    zeros16_f = jnp.zeros((16,), jnp.float32)

    @pl.loop(0, B)
    def _zval(j):
        vals[j] = zeros16_f

    r0 = pl.multiple_of(s * RPS, 8)
    for t in range((RPS + B - 1) // B):
        nr = min(B, RPS - t * B)
        pltpu.sync_copy(vals.at[pl.ds(0, nr)],
                        sd_sh.at[pl.ds(r0 + t * B, nr)])

    @pl.loop(0, B)
    def _pval(j):
        vals[j] = pat
    plsc.subcore_barrier()

    ebase = wid * E_PER_W
    rbase = ebase // SUB

    @pl.loop(0, NCHUNK)
    def _chunk(ci):
        rb = pl.multiple_of(rbase + ci * KSUB, 8)
        eb = pl.multiple_of(ebase + ci * B, 8)
        pltpu.sync_copy(dst_hbm.at[pl.ds(rb, KSUB)], dstv)
        pltpu.sync_copy(w_hbm.at[pl.ds(eb, B)], wv)

        @pl.loop(0, B // 16)
        def _mkval(g):
            w16 = wv[pl.ds(g * 16, 16)]
            plsc.store_scatter(vals, [g * 16 + iota, zeros16_i], w16)

        for j in range(KSUB):
            pltpu.sync_copy(vals.at[pl.ds(j * SUB, SUB)],
                            sd_sh.at[dstv.at[j]], add=True)

    plsc.subcore_barrier()
    pltpu.sync_copy(sd_sh.at[pl.ds(r0, RPS)], sd_out.at[c, pl.ds(r0, RPS)])


_SC_CP = pltpu.CompilerParams(needs_layout_passes=False,
                              use_tc_tiling_on_sc=False)


@jax.jit
def _sc_segment_sums(f, src2, dst2, w1):
    mesh = plsc.VectorSubcoreMesh(core_axis_name="c", subcore_axis_name="s")
    acc_k = pl.kernel(
        _acc_body,
        compiler_params=_SC_CP,
        out_type=jax.ShapeDtypeStruct((NC, N1, Q), jnp.float32),
        mesh=mesh,
        scratch_types=[
            pltpu.VMEM((KSUB, SUB), jnp.int32),
            pltpu.VMEM((KSUB, SUB), jnp.int32),
            pltpu.VMEM((B,), jnp.float32),
            pltpu.VMEM((B, Q), jnp.float32),
            pltpu.VMEM_SHARED((N1, Q), jnp.float32),
            pltpu.SemaphoreType.DMA,
        ],
    )
    sd_k = pl.kernel(
        _sd_body,
        compiler_params=_SC_CP,
        out_type=jax.ShapeDtypeStruct((NC, N1, Q), jnp.float32),
        mesh=mesh,
        scratch_types=[
            pltpu.VMEM((KSUB, SUB), jnp.int32),
            pltpu.VMEM((B,), jnp.float32),
            pltpu.VMEM((B, Q), jnp.float32),
            pltpu.VMEM_SHARED((N1, Q), jnp.float32),
            pltpu.SemaphoreType.DMA,
        ],
    )
    return acc_k(f, src2, dst2, w1), sd_k(dst2, w1)


def _combine_body(f_ref, coll_ref, srcterm_ref, acc_ref, sd_ref, xi_ref,
                  out_ref):
    f = f_ref[...]
    a = acc_ref[0] + acc_ref[1]
    sv = sd_ref[0, :, 0:1] + sd_ref[1, :, 0:1]
    deg = sd_ref[0, :, 1:2] + sd_ref[1, :, 1:2]
    deg = jnp.maximum(deg, 1.0)
    xi = xi_ref[...]
    transport = xi * (a - f * sv) / deg
    out_ref[...] = f - DT * (transport - coll_ref[...] + srcterm_ref[...])


@jax.jit
def _tc_combine(f, coll, srcterm, acc, sd, xi):
    R = 1000
    grid = (N_NODES // R,)
    return pl.pallas_call(
        _combine_body,
        grid=grid,
        in_specs=[
            pl.BlockSpec((R, Q), lambda i: (i, 0)),
            pl.BlockSpec((R, Q), lambda i: (i, 0)),
            pl.BlockSpec((R, Q), lambda i: (i, 0)),
            pl.BlockSpec((NC, R, Q), lambda i: (0, i, 0)),
            pl.BlockSpec((NC, R, Q), lambda i: (0, i, 0)),
            pl.BlockSpec((1, Q), lambda i: (0, 0)),
        ],
        out_specs=pl.BlockSpec((R, Q), lambda i: (i, 0)),
        out_shape=jax.ShapeDtypeStruct((N_NODES, Q), jnp.float32),
    )(f, coll, srcterm, acc, sd, xi)


def kernel(f_distribution, collision_term, source_term, edge_index,
           edge_weight, xi_velocities):
    E = edge_weight.shape[0]
    pad = E_PAD - E
    src = jnp.concatenate([edge_index[0], jnp.zeros((pad,), jnp.int32)])
    dst = jnp.concatenate([edge_index[1],
                           jnp.full((pad,), N_NODES, jnp.int32)])
    w = jnp.concatenate([edge_weight, jnp.zeros((pad,), jnp.float32)])
    src2 = src.reshape(E_PAD // SUB, SUB)
    dst2 = dst.reshape(E_PAD // SUB, SUB)
    acc, sd = _sc_segment_sums(f_distribution, src2, dst2, w)
    return _tc_combine(f_distribution, collision_term, source_term, acc, sd,
                       xi_velocities.reshape(1, Q))


# speedup vs baseline: 42.0393x; 1.0719x over previous
import dataclasses
import functools

import jax
import jax.numpy as jnp
from jax import lax
from jax.experimental import pallas as pl
from jax.experimental.pallas import tpu as pltpu
from jax.experimental.pallas import tpu_sc as plsc

N_NODES = 100000
Q = 16
DT = 0.1

NC = 2
NS = 16
NW = NC * NS
SUB = 128
B = 512
KSUB = B // SUB

N1 = 100352
RPS = N1 // NS

E_PAD = 3211264
E_PER_W = E_PAD // NW
NCHUNK = E_PER_W // B


def _acc_body(f_hbm, src_hbm, dst_hbm, w_hbm,
              acc_out,
              srcv, dstv, wv, rows0, rows1,
              acc_sh, sem_g, sem_sc):
    c = lax.axis_index("c")
    s = lax.axis_index("s")
    wid = s * NC + c

    zeros16_i = jnp.zeros((16,), jnp.int32)
    zeros16_f = jnp.zeros((16,), jnp.float32)

    @pl.loop(0, B)
    def _zrow(j):
        rows0[j] = zeros16_f

    r0 = pl.multiple_of(s * RPS, 8)
    for t in range((RPS + B - 1) // B):
        nr = min(B, RPS - t * B)
        pltpu.sync_copy(rows0.at[pl.ds(0, nr)],
                        acc_sh.at[pl.ds(r0 + t * B, nr)])
    plsc.subcore_barrier()

    ebase = wid * E_PER_W
    rbase = ebase // SUB

    def fire_gathers(rows, half):
        return [
            pltpu.async_copy(f_hbm.at[srcv.at[half * KSUB + j]],
                             rows.at[pl.ds(j * SUB, SUB)], sem_g)
            for j in range(KSUB)
        ]

    def fire_scatters(rows, half):
        return [
            pltpu.async_copy(rows.at[pl.ds(j * SUB, SUB)],
                             acc_sh.at[dstv.at[half * KSUB + j]], sem_sc,
                             add=True)
            for j in range(KSUB)
        ]

    def multiply(rows, half):
        off = half * B

        @pl.loop(0, B)
        def _mul(j):
            wb = plsc.load_gather(wv, [zeros16_i + (off + j)])
            rows[j] = rows[j] * wb

    @pl.loop(0, NCHUNK, step=2)
    def _pipe(i):
        rb = pl.multiple_of(rbase + i * KSUB, 8)
        eb = pl.multiple_of(ebase + i * B, 8)
        pltpu.sync_copy(src_hbm.at[pl.ds(rb, 2 * KSUB)], srcv)
        pltpu.sync_copy(dst_hbm.at[pl.ds(rb, 2 * KSUB)], dstv)
        pltpu.sync_copy(w_hbm.at[pl.ds(eb, 2 * B)], wv)
        g0 = fire_gathers(rows0, 0)
        g1 = fire_gathers(rows1, 1)
        for h in g0:
            h.wait()
        multiply(rows0, 0)
        s0 = fire_scatters(rows0, 0)
        for h in g1:
            h.wait()
        multiply(rows1, 1)
        for h in s0:
            h.wait()
        s1 = fire_scatters(rows1, 1)
        for h in s1:
            h.wait()

    plsc.subcore_barrier()
    pltpu.sync_copy(acc_sh.at[pl.ds(r0, RPS)], acc_out.at[c, pl.ds(r0, RPS)])


def _sd_body(dst_hbm, w_hbm,
             sd_out,
             dstv, wv, vals, sd_sh, gsem):
    c = lax.axis_index("c")
    s = lax.axis_index("s")
    wid = s * NC + c

    iota = lax.iota(jnp.int32, 16)
    pat = jnp.where(iota == 1, 1.0, 0.0).astype(jnp.float32)
    zeros16_i = jnp.zeros((16,), jnp.int32)
    zeros16_f = jnp.zeros((16,), jnp.float32)

    @pl.loop(0, B)
    def _zval(j):
        vals[j] = zeros16_f

    r0 = pl.multiple_of(s * RPS, 8)
    for t in range((RPS + B - 1) // B):
        nr = min(B, RPS - t * B)
        pltpu.sync_copy(vals.at[pl.ds(0, nr)],
                        sd_sh.at[pl.ds(r0 + t * B, nr)])

    @pl.loop(0, B)
    def _pval(j):
        vals[j] = pat
    plsc.subcore_barrier()

    ebase = wid * E_PER_W
    rbase = ebase // SUB

    @pl.loop(0, NCHUNK, step=2)
    def _chunk(ci):
        rb = pl.multiple_of(rbase + ci * KSUB, 8)
        eb = pl.multiple_of(ebase + ci * B, 8)
        pltpu.sync_copy(dst_hbm.at[pl.ds(rb, 2 * KSUB)], dstv)
        pltpu.sync_copy(w_hbm.at[pl.ds(eb, 2 * B)], wv)

        for half in range(2):
            @pl.loop(0, B // 16)
            def _mkval(g):
                w16 = wv[pl.ds(half * B + g * 16, 16)]
                plsc.store_scatter(vals, [g * 16 + iota, zeros16_i], w16)

            for j in range(KSUB):
                pltpu.sync_copy(vals.at[pl.ds(j * SUB, SUB)],
                                sd_sh.at[dstv.at[half * KSUB + j]], add=True)

    plsc.subcore_barrier()
    pltpu.sync_copy(sd_sh.at[pl.ds(r0, RPS)], sd_out.at[c, pl.ds(r0, RPS)])


_SC_CP = pltpu.CompilerParams(needs_layout_passes=False,
                              use_tc_tiling_on_sc=False)


@jax.jit
def _sc_segment_sums(f, src2, dst2, w1):
    mesh = plsc.VectorSubcoreMesh(core_axis_name="c", subcore_axis_name="s")
    acc_k = pl.kernel(
        _acc_body,
        compiler_params=_SC_CP,
        out_type=jax.ShapeDtypeStruct((NC, N1, Q), jnp.float32),
        mesh=mesh,
        scratch_types=[
            pltpu.VMEM((2 * KSUB, SUB), jnp.int32),
            pltpu.VMEM((2 * KSUB, SUB), jnp.int32),
            pltpu.VMEM((2 * B,), jnp.float32),
            pltpu.VMEM((B, Q), jnp.float32),
            pltpu.VMEM((B, Q), jnp.float32),
            pltpu.VMEM_SHARED((N1, Q), jnp.float32),
            pltpu.SemaphoreType.DMA,
            pltpu.SemaphoreType.DMA,
        ],
    )
    sd_k = pl.kernel(
        _sd_body,
        compiler_params=_SC_CP,
        out_type=jax.ShapeDtypeStruct((NC, N1, Q), jnp.float32),
        mesh=mesh,
        scratch_types=[
            pltpu.VMEM((2 * KSUB, SUB), jnp.int32),
            pltpu.VMEM((2 * B,), jnp.float32),
            pltpu.VMEM((B, Q), jnp.float32),
            pltpu.VMEM_SHARED((N1, Q), jnp.float32),
            pltpu.SemaphoreType.DMA,
        ],
    )
    return acc_k(f, src2, dst2, w1), sd_k(dst2, w1)


def _combine_body(f_ref, coll_ref, srcterm_ref, acc_ref, sd_ref, xi_ref,
                  out_ref):
    f = f_ref[...]
    a = acc_ref[0] + acc_ref[1]
    sv = sd_ref[0, :, 0:1] + sd_ref[1, :, 0:1]
    deg = sd_ref[0, :, 1:2] + sd_ref[1, :, 1:2]
    deg = jnp.maximum(deg, 1.0)
    xi = xi_ref[...]
    transport = xi * (a - f * sv) / deg
    out_ref[...] = f - DT * (transport - coll_ref[...] + srcterm_ref[...])


@jax.jit
def _tc_combine(f, coll, srcterm, acc, sd, xi):
    R = 1000
    grid = (N_NODES // R,)
    return pl.pallas_call(
        _combine_body,
        grid=grid,
        in_specs=[
            pl.BlockSpec((R, Q), lambda i: (i, 0)),
            pl.BlockSpec((R, Q), lambda i: (i, 0)),
            pl.BlockSpec((R, Q), lambda i: (i, 0)),
            pl.BlockSpec((NC, R, Q), lambda i: (0, i, 0)),
            pl.BlockSpec((NC, R, Q), lambda i: (0, i, 0)),
            pl.BlockSpec((1, Q), lambda i: (0, 0)),
        ],
        out_specs=pl.BlockSpec((R, Q), lambda i: (i, 0)),
        out_shape=jax.ShapeDtypeStruct((N_NODES, Q), jnp.float32),
    )(f, coll, srcterm, acc, sd, xi)


def kernel(f_distribution, collision_term, source_term, edge_index,
           edge_weight, xi_velocities):
    E = edge_weight.shape[0]
    pad = E_PAD - E
    src = jnp.concatenate([edge_index[0], jnp.zeros((pad,), jnp.int32)])
    dst = jnp.concatenate([edge_index[1],
                           jnp.full((pad,), N_NODES, jnp.int32)])
    w = jnp.concatenate([edge_weight, jnp.zeros((pad,), jnp.float32)])
    src2 = src.reshape(E_PAD // SUB, SUB)
    dst2 = dst.reshape(E_PAD // SUB, SUB)
    acc, sd = _sc_segment_sums(f_distribution, src2, dst2, w)
    return _tc_combine(f_distribution, collision_term, source_term, acc, sd,
                       xi_velocities.reshape(1, Q))


# speedup vs baseline: 59.2772x; 1.4100x over previous
import dataclasses
import functools

import jax
import jax.numpy as jnp
from jax import lax
from jax.experimental import pallas as pl
from jax.experimental.pallas import tpu as pltpu
from jax.experimental.pallas import tpu_sc as plsc

N_NODES = 100000
Q = 16
DT = 0.1

NC = 2
NS = 16
NW = NC * NS
SUB = 128
B = 512
KSUB = B // SUB

N1 = 100352
RPS = N1 // NS

E_PAD = 3211264
E_PER_W = E_PAD // NW
NCHUNK = E_PER_W // B


def _acc_body(f_hbm, src_hbm, dst_hbm, w_hbm,
              acc_out,
              srcv, dstv, wv, rows0, rows1,
              acc_sh, sem_g, sem_sc):
    c = lax.axis_index("c")
    s = lax.axis_index("s")
    wid = s * NC + c

    zeros16_i = jnp.zeros((16,), jnp.int32)
    zeros16_f = jnp.zeros((16,), jnp.float32)

    @pl.loop(0, B)
    def _zrow(j):
        rows0[j] = zeros16_f

    r0 = pl.multiple_of(s * RPS, 8)
    for t in range((RPS + B - 1) // B):
        nr = min(B, RPS - t * B)
        pltpu.sync_copy(rows0.at[pl.ds(0, nr)],
                        acc_sh.at[pl.ds(r0 + t * B, nr)])
    plsc.subcore_barrier()

    ebase = wid * E_PER_W
    rbase = ebase // SUB

    def fire_gathers(rows, half):
        return [
            pltpu.async_copy(f_hbm.at[srcv.at[half * KSUB + j]],
                             rows.at[pl.ds(j * SUB, SUB)], sem_g)
            for j in range(KSUB)
        ]

    def fire_scatters(rows, half):
        return [
            pltpu.async_copy(rows.at[pl.ds(j * SUB, SUB)],
                             acc_sh.at[dstv.at[half * KSUB + j]], sem_sc,
                             add=True)
            for j in range(KSUB)
        ]

    def multiply(rows, half):
        off = half * B

        @plsc.parallel_loop(0, B, unroll=8)
        def _mul(j):
            wb = plsc.load_gather(wv, [zeros16_i + (off + j)])
            rows[j] = rows[j] * wb

    @pl.loop(0, NCHUNK, step=2)
    def _pipe(i):
        rb = pl.multiple_of(rbase + i * KSUB, 8)
        eb = pl.multiple_of(ebase + i * B, 8)
        pltpu.sync_copy(src_hbm.at[pl.ds(rb, 2 * KSUB)], srcv)
        pltpu.sync_copy(dst_hbm.at[pl.ds(rb, 2 * KSUB)], dstv)
        pltpu.sync_copy(w_hbm.at[pl.ds(eb, 2 * B)], wv)
        g0 = fire_gathers(rows0, 0)
        g1 = fire_gathers(rows1, 1)
        for h in g0:
            h.wait()
        multiply(rows0, 0)
        s0 = fire_scatters(rows0, 0)
        for h in g1:
            h.wait()
        multiply(rows1, 1)
        for h in s0:
            h.wait()
        s1 = fire_scatters(rows1, 1)
        for h in s1:
            h.wait()

    plsc.subcore_barrier()
    pltpu.sync_copy(acc_sh.at[pl.ds(r0, RPS)], acc_out.at[c, pl.ds(r0, RPS)])


def _sd_body(dst_hbm, w_hbm,
             sd_out,
             dstv, wv, vals, sd_sh, gsem):
    c = lax.axis_index("c")
    s = lax.axis_index("s")
    wid = s * NC + c

    iota = lax.iota(jnp.int32, 16)
    pat = jnp.where(iota == 1, 1.0, 0.0).astype(jnp.float32)
    zeros16_i = jnp.zeros((16,), jnp.int32)
    zeros16_f = jnp.zeros((16,), jnp.float32)

    @pl.loop(0, B)
    def _zval(j):
        vals[j] = zeros16_f

    r0 = pl.multiple_of(s * RPS, 8)
    for t in range((RPS + B - 1) // B):
        nr = min(B, RPS - t * B)
        pltpu.sync_copy(vals.at[pl.ds(0, nr)],
                        sd_sh.at[pl.ds(r0 + t * B, nr)])

    @pl.loop(0, B)
    def _pval(j):
        vals[j] = pat
    plsc.subcore_barrier()

    ebase = wid * E_PER_W
    rbase = ebase // SUB

    @pl.loop(0, NCHUNK, step=2)
    def _chunk(ci):
        rb = pl.multiple_of(rbase + ci * KSUB, 8)
        eb = pl.multiple_of(ebase + ci * B, 8)
        pltpu.sync_copy(dst_hbm.at[pl.ds(rb, 2 * KSUB)], dstv)
        pltpu.sync_copy(w_hbm.at[pl.ds(eb, 2 * B)], wv)

        for half in range(2):
            @pl.loop(0, B // 16)
            def _mkval(g):
                w16 = wv[pl.ds(half * B + g * 16, 16)]
                plsc.store_scatter(vals, [g * 16 + iota, zeros16_i], w16)

            for j in range(KSUB):
                pltpu.sync_copy(vals.at[pl.ds(j * SUB, SUB)],
                                sd_sh.at[dstv.at[half * KSUB + j]], add=True)

    plsc.subcore_barrier()
    pltpu.sync_copy(sd_sh.at[pl.ds(r0, RPS)], sd_out.at[c, pl.ds(r0, RPS)])


_SC_CP = pltpu.CompilerParams(needs_layout_passes=False,
                              use_tc_tiling_on_sc=False)


@jax.jit
def _sc_segment_sums(f, src2, dst2, w1):
    mesh = plsc.VectorSubcoreMesh(core_axis_name="c", subcore_axis_name="s")
    acc_k = pl.kernel(
        _acc_body,
        compiler_params=_SC_CP,
        out_type=jax.ShapeDtypeStruct((NC, N1, Q), jnp.float32),
        mesh=mesh,
        scratch_types=[
            pltpu.VMEM((2 * KSUB, SUB), jnp.int32),
            pltpu.VMEM((2 * KSUB, SUB), jnp.int32),
            pltpu.VMEM((2 * B,), jnp.float32),
            pltpu.VMEM((B, Q), jnp.float32),
            pltpu.VMEM((B, Q), jnp.float32),
            pltpu.VMEM_SHARED((N1, Q), jnp.float32),
            pltpu.SemaphoreType.DMA,
            pltpu.SemaphoreType.DMA,
        ],
    )
    sd_k = pl.kernel(
        _sd_body,
        compiler_params=_SC_CP,
        out_type=jax.ShapeDtypeStruct((NC, N1, Q), jnp.float32),
        mesh=mesh,
        scratch_types=[
            pltpu.VMEM((2 * KSUB, SUB), jnp.int32),
            pltpu.VMEM((2 * B,), jnp.float32),
            pltpu.VMEM((B, Q), jnp.float32),
            pltpu.VMEM_SHARED((N1, Q), jnp.float32),
            pltpu.SemaphoreType.DMA,
        ],
    )
    return acc_k(f, src2, dst2, w1), sd_k(dst2, w1)


def _combine_body(f_ref, coll_ref, srcterm_ref, acc_ref, sd_ref, xi_ref,
                  out_ref):
    f = f_ref[...]
    a = acc_ref[0] + acc_ref[1]
    sv = sd_ref[0, :, 0:1] + sd_ref[1, :, 0:1]
    deg = sd_ref[0, :, 1:2] + sd_ref[1, :, 1:2]
    deg = jnp.maximum(deg, 1.0)
    xi = xi_ref[...]
    transport = xi * (a - f * sv) / deg
    out_ref[...] = f - DT * (transport - coll_ref[...] + srcterm_ref[...])


@jax.jit
def _tc_combine(f, coll, srcterm, acc, sd, xi):
    R = 1000
    grid = (N_NODES // R,)
    return pl.pallas_call(
        _combine_body,
        grid=grid,
        in_specs=[
            pl.BlockSpec((R, Q), lambda i: (i, 0)),
            pl.BlockSpec((R, Q), lambda i: (i, 0)),
            pl.BlockSpec((R, Q), lambda i: (i, 0)),
            pl.BlockSpec((NC, R, Q), lambda i: (0, i, 0)),
            pl.BlockSpec((NC, R, Q), lambda i: (0, i, 0)),
            pl.BlockSpec((1, Q), lambda i: (0, 0)),
        ],
        out_specs=pl.BlockSpec((R, Q), lambda i: (i, 0)),
        out_shape=jax.ShapeDtypeStruct((N_NODES, Q), jnp.float32),
    )(f, coll, srcterm, acc, sd, xi)


def kernel(f_distribution, collision_term, source_term, edge_index,
           edge_weight, xi_velocities):
    E = edge_weight.shape[0]
    pad = E_PAD - E
    src = jnp.concatenate([edge_index[0], jnp.zeros((pad,), jnp.int32)])
    dst = jnp.concatenate([edge_index[1],
                           jnp.full((pad,), N_NODES, jnp.int32)])
    w = jnp.concatenate([edge_weight, jnp.zeros((pad,), jnp.float32)])
    src2 = src.reshape(E_PAD // SUB, SUB)
    dst2 = dst.reshape(E_PAD // SUB, SUB)
    acc, sd = _sc_segment_sums(f_distribution, src2, dst2, w)
    return _tc_combine(f_distribution, collision_term, source_term, acc, sd,
                       xi_velocities.reshape(1, Q))


# speedup vs baseline: 61.7549x; 1.0418x over previous
import dataclasses
import functools

import jax
import jax.numpy as jnp
from jax import lax
from jax.experimental import pallas as pl
from jax.experimental.pallas import tpu as pltpu
from jax.experimental.pallas import tpu_sc as plsc

N_NODES = 100000
Q = 16
DT = 0.1

NC = 2
NS = 16
NW = NC * NS
SUB = 128
B = 512
KSUB = B // SUB

N1 = 100352
RPS = N1 // NS

E_PAD = 3211264
E_PER_W = E_PAD // NW
NCHUNK = E_PER_W // B


def _acc_body(f_hbm, src_hbm, dst_hbm, w_hbm,
              acc_out,
              srcv, dstv, wv, rows0, rows1,
              acc_sh, sem_g, sem_sc):
    c = lax.axis_index("c")
    s = lax.axis_index("s")
    wid = s * NC + c

    zeros16_i = jnp.zeros((16,), jnp.int32)
    zeros16_f = jnp.zeros((16,), jnp.float32)

    @pl.loop(0, B)
    def _zrow(j):
        rows0[j] = zeros16_f

    r0 = pl.multiple_of(s * RPS, 8)
    for t in range((RPS + B - 1) // B):
        nr = min(B, RPS - t * B)
        pltpu.sync_copy(rows0.at[pl.ds(0, nr)],
                        acc_sh.at[pl.ds(r0 + t * B, nr)])
    plsc.subcore_barrier()

    ebase = wid * E_PER_W
    rbase = ebase // SUB

    def fire_gathers(rows, half):
        return [
            pltpu.async_copy(f_hbm.at[srcv.at[half * KSUB + j]],
                             rows.at[pl.ds(j * SUB, SUB)], sem_g)
            for j in range(KSUB)
        ]

    def fire_scatters(rows, half):
        return [
            pltpu.async_copy(rows.at[pl.ds(j * SUB, SUB)],
                             acc_sh.at[dstv.at[half * KSUB + j]], sem_sc,
                             add=True)
            for j in range(KSUB)
        ]

    def multiply(rows, half):
        off = half * B

        @plsc.parallel_loop(0, B, unroll=8)
        def _mul(j):
            wb = plsc.load_gather(wv, [zeros16_i + (off + j)])
            rows[j] = rows[j] * wb

    @pl.loop(0, NCHUNK, step=2)
    def _pipe(i):
        rb = pl.multiple_of(rbase + i * KSUB, 8)
        eb = pl.multiple_of(ebase + i * B, 8)
        pltpu.sync_copy(src_hbm.at[pl.ds(rb, 2 * KSUB)], srcv)
        pltpu.sync_copy(dst_hbm.at[pl.ds(rb, 2 * KSUB)], dstv)
        pltpu.sync_copy(w_hbm.at[pl.ds(eb, 2 * B)], wv)
        g0 = fire_gathers(rows0, 0)
        g1 = fire_gathers(rows1, 1)
        for h in g0:
            h.wait()
        multiply(rows0, 0)
        s0 = fire_scatters(rows0, 0)
        for h in g1:
            h.wait()
        multiply(rows1, 1)
        for h in s0:
            h.wait()
        s1 = fire_scatters(rows1, 1)
        for h in s1:
            h.wait()

    plsc.subcore_barrier()
    pltpu.sync_copy(acc_sh.at[pl.ds(r0, RPS)], acc_out.at[c, pl.ds(r0, RPS)])


def _sd_body(dst_hbm, w_hbm,
             sd_out,
             dstv, wv, vals0, vals1, sd_sh, sem_sc):
    c = lax.axis_index("c")
    s = lax.axis_index("s")
    wid = s * NC + c

    iota = lax.iota(jnp.int32, 16)
    pat = jnp.where(iota == 1, 1.0, 0.0).astype(jnp.float32)
    zeros16_i = jnp.zeros((16,), jnp.int32)
    zeros16_f = jnp.zeros((16,), jnp.float32)

    @pl.loop(0, B)
    def _zval(j):
        vals0[j] = zeros16_f

    r0 = pl.multiple_of(s * RPS, 8)
    for t in range((RPS + B - 1) // B):
        nr = min(B, RPS - t * B)
        pltpu.sync_copy(vals0.at[pl.ds(0, nr)],
                        sd_sh.at[pl.ds(r0 + t * B, nr)])

    @pl.loop(0, B)
    def _pval(j):
        vals0[j] = pat
        vals1[j] = pat
    plsc.subcore_barrier()

    ebase = wid * E_PER_W
    rbase = ebase // SUB

    def build(vals, half):
        @plsc.parallel_loop(0, B // 16, unroll=4)
        def _mkval(g):
            w16 = wv[pl.ds(half * B + g * 16, 16)]
            plsc.store_scatter(vals, [g * 16 + iota, zeros16_i], w16)

    def fire_scatters(vals, half):
        return [
            pltpu.async_copy(vals.at[pl.ds(j * SUB, SUB)],
                             sd_sh.at[dstv.at[half * KSUB + j]], sem_sc,
                             add=True)
            for j in range(KSUB)
        ]

    @pl.loop(0, NCHUNK, step=2)
    def _chunk(ci):
        rb = pl.multiple_of(rbase + ci * KSUB, 8)
        eb = pl.multiple_of(ebase + ci * B, 8)
        pltpu.sync_copy(dst_hbm.at[pl.ds(rb, 2 * KSUB)], dstv)
        pltpu.sync_copy(w_hbm.at[pl.ds(eb, 2 * B)], wv)

        build(vals0, 0)
        s0 = fire_scatters(vals0, 0)
        build(vals1, 1)
        for h in s0:
            h.wait()
        s1 = fire_scatters(vals1, 1)
        for h in s1:
            h.wait()

    plsc.subcore_barrier()
    pltpu.sync_copy(sd_sh.at[pl.ds(r0, RPS)], sd_out.at[c, pl.ds(r0, RPS)])


_SC_CP = pltpu.CompilerParams(needs_layout_passes=False,
                              use_tc_tiling_on_sc=False)


@jax.jit
def _sc_segment_sums(f, src2, dst2, w1):
    mesh = plsc.VectorSubcoreMesh(core_axis_name="c", subcore_axis_name="s")
    acc_k = pl.kernel(
        _acc_body,
        compiler_params=_SC_CP,
        out_type=jax.ShapeDtypeStruct((NC, N1, Q), jnp.float32),
        mesh=mesh,
        scratch_types=[
            pltpu.VMEM((2 * KSUB, SUB), jnp.int32),
            pltpu.VMEM((2 * KSUB, SUB), jnp.int32),
            pltpu.VMEM((2 * B,), jnp.float32),
            pltpu.VMEM((B, Q), jnp.float32),
            pltpu.VMEM((B, Q), jnp.float32),
            pltpu.VMEM_SHARED((N1, Q), jnp.float32),
            pltpu.SemaphoreType.DMA,
            pltpu.SemaphoreType.DMA,
        ],
    )
    sd_k = pl.kernel(
        _sd_body,
        compiler_params=_SC_CP,
        out_type=jax.ShapeDtypeStruct((NC, N1, Q), jnp.float32),
        mesh=mesh,
        scratch_types=[
            pltpu.VMEM((2 * KSUB, SUB), jnp.int32),
            pltpu.VMEM((2 * B,), jnp.float32),
            pltpu.VMEM((B, Q), jnp.float32),
            pltpu.VMEM((B, Q), jnp.float32),
            pltpu.VMEM_SHARED((N1, Q), jnp.float32),
            pltpu.SemaphoreType.DMA,
        ],
    )
    return acc_k(f, src2, dst2, w1), sd_k(dst2, w1)


def _combine_body(f_ref, coll_ref, srcterm_ref, acc_ref, sd_ref, xi_ref,
                  out_ref):
    f = f_ref[...]
    a = acc_ref[0] + acc_ref[1]
    sv = sd_ref[0, :, 0:1] + sd_ref[1, :, 0:1]
    deg = sd_ref[0, :, 1:2] + sd_ref[1, :, 1:2]
    deg = jnp.maximum(deg, 1.0)
    xi = xi_ref[...]
    transport = xi * (a - f * sv) / deg
    out_ref[...] = f - DT * (transport - coll_ref[...] + srcterm_ref[...])


@jax.jit
def _tc_combine(f, coll, srcterm, acc, sd, xi):
    R = 1000
    grid = (N_NODES // R,)
    return pl.pallas_call(
        _combine_body,
        grid=grid,
        in_specs=[
            pl.BlockSpec((R, Q), lambda i: (i, 0)),
            pl.BlockSpec((R, Q), lambda i: (i, 0)),
            pl.BlockSpec((R, Q), lambda i: (i, 0)),
            pl.BlockSpec((NC, R, Q), lambda i: (0, i, 0)),
            pl.BlockSpec((NC, R, Q), lambda i: (0, i, 0)),
            pl.BlockSpec((1, Q), lambda i: (0, 0)),
        ],
        out_specs=pl.BlockSpec((R, Q), lambda i: (i, 0)),
        out_shape=jax.ShapeDtypeStruct((N_NODES, Q), jnp.float32),
    )(f, coll, srcterm, acc, sd, xi)


def kernel(f_distribution, collision_term, source_term, edge_index,
           edge_weight, xi_velocities):
    E = edge_weight.shape[0]
    pad = E_PAD - E
    src = jnp.concatenate([edge_index[0], jnp.zeros((pad,), jnp.int32)])
    dst = jnp.concatenate([edge_index[1],
                           jnp.full((pad,), N_NODES, jnp.int32)])
    w = jnp.concatenate([edge_weight, jnp.zeros((pad,), jnp.float32)])
    src2 = src.reshape(E_PAD // SUB, SUB)
    dst2 = dst.reshape(E_PAD // SUB, SUB)
    acc, sd = _sc_segment_sums(f_distribution, src2, dst2, w)
    return _tc_combine(f_distribution, collision_term, source_term, acc, sd,
                       xi_velocities.reshape(1, Q))
